# Initial kernel scaffold; baseline (speedup 1.0000x reference)
import jax
import jax.numpy as jnp
from jax.experimental import pallas as pl


def kernel(drug_feature, drug_adj, ibatch, params):
    raise NotImplementedError("write your pallas kernel here")


# speedup vs baseline: 5.8303x; 5.8303x over previous
import functools

import jax
import jax.numpy as jnp
from jax import lax
from jax.experimental import pallas as pl
from jax.experimental.pallas import tpu as pltpu
from jax.experimental.pallas import tpu_sc as plsc

N = 10000
D = 128
G = 64

NC = 2
NS = 16
LANES = 16
K = 128
NP = 10240
HALF = NP // 2
JROWS = 128
ACC_ROWS = HALF + JROWS
ZBLKS = ACC_ROWS // K
OUT_RPT = HALF // NS
PAD_ROW = N

_HIGH = jax.lax.Precision.HIGHEST
_f32 = jnp.float32


def _fill(ref, value):
  v = jnp.full((LANES,), value, _f32)

  @pl.loop(0, K)
  def _(r):
    for cg in range(D // LANES):
      ref[r, pl.ds(cg * LANES, LANES)] = v


def _zero_acc(acc, zsrc, s):
  for bi in range(-(-ZBLKS // NS)):
    b = bi * NS + s

    @pl.when(b < ZBLKS)
    def _():
      pltpu.sync_copy(zsrc, acc.at[pl.ds(b * K, K)])


def _make_edge_scatter(with_deg, nblk):
  mesh = plsc.VectorSubcoreMesh(core_axis_name="c", subcore_axis_name="s")
  out_type = jax.ShapeDtypeStruct((NP, D), _f32)
  if with_deg:
    out_type = [out_type, jax.ShapeDtypeStruct((NP, D), _f32)]
  scratch = [
      pltpu.VMEM((nblk, K), jnp.int32),
      pltpu.VMEM((nblk, K), jnp.int32),
      pltpu.VMEM((2, K, D), _f32),
      pltpu.VMEM_SHARED((ACC_ROWS, D), _f32),
      pltpu.SemaphoreType.DMA,
      pltpu.SemaphoreType.DMA,
  ]

  def body(feat, srcp, dstp, *rest):
    if with_deg:
      out, deg_out, srcv, dstv, rows, acc, sem0, sem1 = rest
    else:
      out, srcv, dstv, rows, acc, sem0, sem1 = rest
    c = lax.axis_index("c")
    s = lax.axis_index("s")

    pltpu.sync_copy(srcp.at[pl.ds(s * nblk, nblk)], srcv)
    pltpu.sync_copy(dstp.at[pl.ds(s * nblk, nblk)], dstv)

    base = c * HALF

    @pl.loop(0, nblk)
    def _(r):
      for cg in range(K // LANES):
        dv = dstv[r, pl.ds(cg * LANES, LANES)]
        loc = dv - base
        inr = (loc >= 0) & (loc < HALF)
        junk = HALF + jnp.bitwise_and(dv, JROWS - 1)
        dstv[r, pl.ds(cg * LANES, LANES)] = jnp.where(inr, loc, junk)

    _fill(rows.at[0], 0.0)

    def drain(out_ref):
      plsc.subcore_barrier()
      pltpu.sync_copy(acc.at[pl.ds(s * OUT_RPT, OUT_RPT)],
                      out_ref.at[pl.ds(base + s * OUT_RPT, OUT_RPT)])
      plsc.subcore_barrier()

    if with_deg:
      _zero_acc(acc, rows.at[0], s)
      _fill(rows.at[1], 1.0)
      plsc.subcore_barrier()

      @pl.loop(0, nblk)
      def _(j):
        pltpu.sync_copy(rows.at[1], acc.at[dstv.at[j]], add=True)

      drain(deg_out)

    _zero_acc(acc, rows.at[0], s)
    plsc.subcore_barrier()

    @pl.loop(0, nblk // 2)
    def _(i):
      j0 = 2 * i
      j1 = j0 + 1
      cp0 = pltpu.async_copy(feat.at[srcv.at[j0]], rows.at[0], sem0)
      cp1 = pltpu.async_copy(feat.at[srcv.at[j1]], rows.at[1], sem1)
      cp0.wait()
      pltpu.sync_copy(rows.at[0], acc.at[dstv.at[j0]], add=True)
      cp1.wait()
      pltpu.sync_copy(rows.at[1], acc.at[dstv.at[j1]], add=True)

    drain(out)

  return pl.kernel(body, out_type=out_type, mesh=mesh, scratch_types=scratch)


def _bn(x, g, b):
  m = jnp.mean(x, axis=0, keepdims=True)
  v = jnp.mean((x - m) ** 2, axis=0, keepdims=True)
  return (x - m) / jnp.sqrt(v + 1e-5) * g + b


def _bn0_body(x_ref, g_ref, b_ref, o_ref):
  h = _bn(x_ref[...], g_ref[...], b_ref[...])
  o_ref[...] = jnp.concatenate([h, jnp.zeros((NP - N, D), _f32)], axis=0)


_bn0_call = pl.pallas_call(
    _bn0_body, out_shape=jax.ShapeDtypeStruct((NP, D), _f32))


def _mlp_body(layer, *refs):
  if layer == 1:
    (h_ref, a_ref, degp_ref, wa, ba, wb, bb, g_ref, b_ref, pw, pw4,
     h_out, xw_out, xw4_out, dinv_out) = refs
  else:
    (h_ref, a_ref, wa, ba, wb, bb, g_ref, b_ref, pw, pw4, xw4p_ref,
     h_out, xw_out, xw4_out) = refs

  t = h_ref[:N] + a_ref[:N]
  u = jnp.maximum(jnp.dot(t, wa[...]) + ba[...], 0.0)
  r = jnp.maximum(jnp.dot(u, wb[...]) + bb[...], 0.0)
  hn = _bn(r, g_ref[...], b_ref[...])
  h_out[...] = jnp.concatenate([hn, jnp.zeros((NP - N, D), _f32)], axis=0)

  zpad = jnp.zeros((NP - N,), _f32)
  xw = jnp.dot(hn, pw[...])[:, 0]
  xw_out[...] = jnp.concatenate([xw, zpad])[None, :]
  xw4 = jnp.dot(hn, pw4[...])[:, 0]
  if layer != 1:
    xw4 = xw4 + xw4p_ref[0, :N]
  xw4_out[...] = jnp.concatenate([xw4, zpad])[None, :]

  if layer == 1:
    deg = degp_ref[:N, 0] + 1.0
    dinv = deg ** -0.5
    dinv_out[...] = jnp.concatenate([dinv, zpad])[None, :]


def _mlp_call(layer):
  outs = [jax.ShapeDtypeStruct((NP, D), _f32),
          jax.ShapeDtypeStruct((1, NP), _f32),
          jax.ShapeDtypeStruct((1, NP), _f32)]
  if layer == 1:
    outs.append(jax.ShapeDtypeStruct((1, NP), _f32))
  return pl.pallas_call(functools.partial(_mlp_body, layer), out_shape=outs)


def _y_body(xw1_ref, xw2_ref, xw3_ref, xw4_ref, dinv_ref, y_out):
  dinv = dinv_ref[...]
  cols = [xw1_ref[...] * dinv, xw2_ref[...] * dinv,
          xw3_ref[...] * dinv, xw4_ref[...] * dinv]
  y = jnp.concatenate(cols, axis=0).T
  y_out[...] = jnp.concatenate([y, jnp.zeros((NP, D - 4), _f32)], axis=1)


_y_call = pl.pallas_call(
    _y_body, out_shape=jax.ShapeDtypeStruct((NP, D), _f32))


def _pool_body(h1_ref, h2_ref, h3_ref, y_ref, nbr_ref, dinv_ref, batch_ref,
               pb_ref, local_out, g4_out):
  dinv = dinv_ref[0, :N]
  gi = lax.broadcasted_iota(jnp.int32, (G, N), 0)
  ind = (batch_ref[...] == gi).astype(_f32)
  hs = (h1_ref[:N], h2_ref[:N], h3_ref[:N])

  def attn(k):
    sc = dinv * nbr_ref[:N, k] + y_ref[:N, k] * dinv + pb_ref[0, k]
    s = jnp.tanh(sc)
    smax = jnp.max(jnp.where(ind > 0, s[None, :], -1e30), axis=1)
    smb = jnp.dot(smax[None, :], ind, precision=_HIGH)[0]
    e = jnp.exp(s - smb)
    z = jnp.dot(ind, e[:, None], precision=_HIGH)[:, 0]
    zb = jnp.dot(z[None, :], ind, precision=_HIGH)[0]
    return e / (zb + 1e-16)

  gs = []
  for k in range(3):
    a = attn(k)
    gs.append(jnp.dot(ind, hs[k] * a[:, None], precision=_HIGH))
  local_out[...] = jnp.concatenate(gs, axis=1)

  a4 = attn(3)
  g4 = [jnp.dot(ind, h * a4[:, None], precision=_HIGH) for h in hs]
  g4_out[...] = jnp.concatenate(g4, axis=1)


_pool_call = pl.pallas_call(
    _pool_body,
    out_shape=[jax.ShapeDtypeStruct((G, 3 * D), _f32),
               jax.ShapeDtypeStruct((G, 3 * D), _f32)])


def kernel(drug_feature, drug_adj, ibatch, params):
  p = params
  src = drug_adj[0].astype(jnp.int32)
  dst = drug_adj[1].astype(jnp.int32)
  e = src.shape[0]
  nblk = -(-e // (NS * K))
  nblk = -(-nblk // 8) * 8
  ep = NS * nblk * K
  pad = jnp.full((ep - e,), PAD_ROW, jnp.int32)
  srcp = jnp.concatenate([src, pad]).reshape(NS * nblk, K)
  dstp = jnp.concatenate([dst, pad]).reshape(NS * nblk, K)

  agg_deg = _make_edge_scatter(True, nblk)
  agg = _make_edge_scatter(False, nblk)

  row = lambda a: a.reshape(1, -1)
  pw4 = p['pw4']

  h0 = _bn0_call(drug_feature, row(p['bn0_g']), row(p['bn0_b']))
  a1, degp = agg_deg(h0, srcp, dstp)
  h1, xw1, xw4a, dinv = _mlp_call(1)(
      h0, a1, degp, p['w1a'], row(p['b1a']), p['w1b'], row(p['b1b']),
      row(p['bn1_g']), row(p['bn1_b']), p['pw1'],
      pw4[:D])
  a2 = agg(h1, srcp, dstp)
  h2, xw2, xw4b = _mlp_call(2)(
      h1, a2, p['w2a'], row(p['b2a']), p['w2b'], row(p['b2b']),
      row(p['bn2_g']), row(p['bn2_b']), p['pw2'],
      pw4[D:2 * D], xw4a)
  a3 = agg(h2, srcp, dstp)
  h3, xw3, xw4c = _mlp_call(3)(
      h2, a3, p['w3a'], row(p['b3a']), p['w3b'], row(p['b3b']),
      row(p['bn3_g']), row(p['bn3_b']), p['pw3'],
      pw4[2 * D:], xw4b)
  y = _y_call(xw1, xw2, xw3, xw4c, dinv)
  nbr = agg(y, srcp, dstp)

  pb = jnp.concatenate([p['pb1'], p['pb2'], p['pb3'], p['pb4']])
  local, g4 = _pool_call(h1, h2, h3, y, nbr, dinv, row(ibatch),
                         row(pb))
  return (local, g4)


# speedup vs baseline: 6.2375x; 1.0698x over previous
import functools

import jax
import jax.numpy as jnp
from jax import lax
from jax.experimental import pallas as pl
from jax.experimental.pallas import tpu as pltpu
from jax.experimental.pallas import tpu_sc as plsc

N = 10000
D = 128
G = 64

NC = 2
NS = 16
LANES = 16
K = 128
NP = 10240
HALF = NP // 2
JROWS = 128
ACC_ROWS = HALF + JROWS
ZBLKS = ACC_ROWS // K
OUT_RPT = HALF // NS
PAD_ROW = N
NBUF = 2
MB = 1
MROWS = MB * K

_HIGH = jax.lax.Precision.HIGHEST
_f32 = jnp.float32


def _fill(ref, nrows, value):
  v = jnp.full((LANES,), value, _f32)

  @pl.loop(0, nrows)
  def _(r):
    for cg in range(D // LANES):
      ref[r, pl.ds(cg * LANES, LANES)] = v


def _zero_acc(acc, zsrc, s):
  for bi in range(-(-ZBLKS // NS)):
    b = bi * NS + s

    @pl.when(b < ZBLKS)
    def _():
      pltpu.sync_copy(zsrc, acc.at[pl.ds(b * K, K)])


def _make_edge_scatter(with_deg, nblk):
  mesh = plsc.VectorSubcoreMesh(core_axis_name="c", subcore_axis_name="s")
  out_type = jax.ShapeDtypeStruct((NP, D), _f32)
  if with_deg:
    out_type = [out_type, jax.ShapeDtypeStruct((NP, D), _f32)]
  scratch = [
      pltpu.VMEM((nblk // MB, MROWS), jnp.int32),
      pltpu.VMEM((nblk // MB, MROWS), jnp.int32),
      pltpu.VMEM((NBUF, MROWS, D), _f32),
      pltpu.VMEM_SHARED((ACC_ROWS, D), _f32),
      pltpu.SemaphoreType.DMA((NBUF,)),
      pltpu.SemaphoreType.DMA((NBUF,)),
  ]
  nblkm = nblk // MB

  def body(feat, srcp, dstp, *rest):
    if with_deg:
      out, deg_out, srcv, dstv, rows, acc, gsem, ssem = rest
    else:
      out, srcv, dstv, rows, acc, gsem, ssem = rest
    c = lax.axis_index("c")
    s = lax.axis_index("s")

    pltpu.sync_copy(srcp.at[pl.ds(s * nblkm, nblkm)], srcv)
    pltpu.sync_copy(dstp.at[pl.ds(s * nblkm, nblkm)], dstv)

    base = c * HALF

    @pl.loop(0, nblkm)
    def _(r):
      for cg in range(MROWS // LANES):
        dv = dstv[r, pl.ds(cg * LANES, LANES)]
        loc = dv - base
        inr = (loc >= 0) & (loc < HALF)
        junk = HALF + jnp.bitwise_and(dv, JROWS - 1)
        dstv[r, pl.ds(cg * LANES, LANES)] = jnp.where(inr, loc, junk)

    _fill(rows.at[0, pl.ds(0, K)], K, 0.0)

    def drain(out_ref):
      plsc.subcore_barrier()
      pltpu.sync_copy(acc.at[pl.ds(s * OUT_RPT, OUT_RPT)],
                      out_ref.at[pl.ds(base + s * OUT_RPT, OUT_RPT)])
      plsc.subcore_barrier()

    def wait_scatter(src_b, sem_b, j):
      pltpu.make_async_copy(rows.at[src_b], acc.at[dstv.at[j]],
                            ssem.at[sem_b]).wait()

    if with_deg:
      _zero_acc(acc, rows.at[0, pl.ds(0, K)], s)
      _fill(rows.at[1], MROWS, 1.0)
      plsc.subcore_barrier()

      @pl.loop(0, nblkm // NBUF)
      def _(g):
        for b in range(NBUF):
          j = g * NBUF + b

          @pl.when(j >= NBUF)
          def _():
            wait_scatter(1, b, j - NBUF)

          pltpu.async_copy(rows.at[1], acc.at[dstv.at[j]], ssem.at[b],
                           add=True)

      for b in range(NBUF):
        wait_scatter(1, b, nblkm - NBUF + b)

      drain(deg_out)

    _zero_acc(acc, rows.at[0, pl.ds(0, K)], s)
    plsc.subcore_barrier()

    pltpu.async_copy(feat.at[srcv.at[0]], rows.at[0], gsem.at[0])

    @pl.loop(0, nblkm // 2)
    def _(g):
      for b in range(2):
        j = 2 * g + b
        bg = 1 - b

        @pl.when((j >= 1) & (j + 1 < nblkm))
        def _():
          wait_scatter(bg, bg, j - 1)

        @pl.when(j + 1 < nblkm)
        def _():
          pltpu.async_copy(feat.at[srcv.at[j + 1]], rows.at[bg], gsem.at[bg])

        pltpu.make_async_copy(feat.at[srcv.at[j]], rows.at[b],
                              gsem.at[b]).wait()
        pltpu.async_copy(rows.at[b], acc.at[dstv.at[j]], ssem.at[b], add=True)

    for b in range(2):
      wait_scatter(b, b, nblkm - 2 + b)

    drain(out)

  return pl.kernel(body, out_type=out_type, mesh=mesh, scratch_types=scratch)


def _bn(x, g, b):
  m = jnp.mean(x, axis=0, keepdims=True)
  v = jnp.mean((x - m) ** 2, axis=0, keepdims=True)
  return (x - m) / jnp.sqrt(v + 1e-5) * g + b


def _bn0_body(x_ref, g_ref, b_ref, o_ref):
  h = _bn(x_ref[...], g_ref[...], b_ref[...])
  o_ref[...] = jnp.concatenate([h, jnp.zeros((NP - N, D), _f32)], axis=0)


_bn0_call = pl.pallas_call(
    _bn0_body, out_shape=jax.ShapeDtypeStruct((NP, D), _f32))


def _mlp_body(layer, *refs):
  if layer == 1:
    (h_ref, a_ref, degp_ref, wa, ba, wb, bb, g_ref, b_ref, pw, pw4,
     h_out, xw_out, xw4_out, dinv_out) = refs
  else:
    (h_ref, a_ref, wa, ba, wb, bb, g_ref, b_ref, pw, pw4, xw4p_ref,
     h_out, xw_out, xw4_out) = refs

  t = h_ref[:N] + a_ref[:N]
  u = jnp.maximum(jnp.dot(t, wa[...]) + ba[...], 0.0)
  r = jnp.maximum(jnp.dot(u, wb[...]) + bb[...], 0.0)
  hn = _bn(r, g_ref[...], b_ref[...])
  h_out[...] = jnp.concatenate([hn, jnp.zeros((NP - N, D), _f32)], axis=0)

  zpad = jnp.zeros((NP - N,), _f32)
  xw = jnp.dot(hn, pw[...])[:, 0]
  xw_out[...] = jnp.concatenate([xw, zpad])[None, :]
  xw4 = jnp.dot(hn, pw4[...])[:, 0]
  if layer != 1:
    xw4 = xw4 + xw4p_ref[0, :N]
  xw4_out[...] = jnp.concatenate([xw4, zpad])[None, :]

  if layer == 1:
    deg = degp_ref[:N, 0] + 1.0
    dinv = deg ** -0.5
    dinv_out[...] = jnp.concatenate([dinv, zpad])[None, :]


def _mlp_call(layer):
  outs = [jax.ShapeDtypeStruct((NP, D), _f32),
          jax.ShapeDtypeStruct((1, NP), _f32),
          jax.ShapeDtypeStruct((1, NP), _f32)]
  if layer == 1:
    outs.append(jax.ShapeDtypeStruct((1, NP), _f32))
  return pl.pallas_call(functools.partial(_mlp_body, layer), out_shape=outs)


def _y_body(xw1_ref, xw2_ref, xw3_ref, xw4_ref, dinv_ref, y_out):
  dinv = dinv_ref[...]
  cols = [xw1_ref[...] * dinv, xw2_ref[...] * dinv,
          xw3_ref[...] * dinv, xw4_ref[...] * dinv]
  y = jnp.concatenate(cols, axis=0).T
  y_out[...] = jnp.concatenate([y, jnp.zeros((NP, D - 4), _f32)], axis=1)


_y_call = pl.pallas_call(
    _y_body, out_shape=jax.ShapeDtypeStruct((NP, D), _f32))


def _pool_body(h1_ref, h2_ref, h3_ref, y_ref, nbr_ref, dinv_ref, batch_ref,
               pb_ref, local_out, g4_out):
  dinv = dinv_ref[0, :N]
  gi = lax.broadcasted_iota(jnp.int32, (G, N), 0)
  ind = (batch_ref[...] == gi).astype(_f32)
  hs = (h1_ref[:N], h2_ref[:N], h3_ref[:N])

  def attn(k):
    sc = dinv * nbr_ref[:N, k] + y_ref[:N, k] * dinv + pb_ref[0, k]
    s = jnp.tanh(sc)
    smax = jnp.max(jnp.where(ind > 0, s[None, :], -1e30), axis=1)
    smb = jnp.dot(smax[None, :], ind, precision=_HIGH)[0]
    e = jnp.exp(s - smb)
    z = jnp.dot(ind, e[:, None], precision=_HIGH)[:, 0]
    zb = jnp.dot(z[None, :], ind, precision=_HIGH)[0]
    return e / (zb + 1e-16)

  gs = []
  for k in range(3):
    a = attn(k)
    gs.append(jnp.dot(ind, hs[k] * a[:, None], precision=_HIGH))
  local_out[...] = jnp.concatenate(gs, axis=1)

  a4 = attn(3)
  g4 = [jnp.dot(ind, h * a4[:, None], precision=_HIGH) for h in hs]
  g4_out[...] = jnp.concatenate(g4, axis=1)


_pool_call = pl.pallas_call(
    _pool_body,
    out_shape=[jax.ShapeDtypeStruct((G, 3 * D), _f32),
               jax.ShapeDtypeStruct((G, 3 * D), _f32)])


def kernel(drug_feature, drug_adj, ibatch, params):
  p = params
  src = drug_adj[0].astype(jnp.int32)
  dst = drug_adj[1].astype(jnp.int32)
  e = src.shape[0]
  nblk = -(-e // (NS * K))
  nblk = -(-nblk // 8) * 8
  ep = NS * nblk * K
  pad = jnp.full((ep - e,), PAD_ROW, jnp.int32)
  srcp = jnp.concatenate([src, pad]).reshape(NS * nblk // MB, MROWS)
  dstp = jnp.concatenate([dst, pad]).reshape(NS * nblk // MB, MROWS)

  agg_deg = _make_edge_scatter(True, nblk)
  agg = _make_edge_scatter(False, nblk)

  row = lambda a: a.reshape(1, -1)
  pw4 = p['pw4']

  h0 = _bn0_call(drug_feature, row(p['bn0_g']), row(p['bn0_b']))
  a1, degp = agg_deg(h0, srcp, dstp)
  h1, xw1, xw4a, dinv = _mlp_call(1)(
      h0, a1, degp, p['w1a'], row(p['b1a']), p['w1b'], row(p['b1b']),
      row(p['bn1_g']), row(p['bn1_b']), p['pw1'],
      pw4[:D])
  a2 = agg(h1, srcp, dstp)
  h2, xw2, xw4b = _mlp_call(2)(
      h1, a2, p['w2a'], row(p['b2a']), p['w2b'], row(p['b2b']),
      row(p['bn2_g']), row(p['bn2_b']), p['pw2'],
      pw4[D:2 * D], xw4a)
  a3 = agg(h2, srcp, dstp)
  h3, xw3, xw4c = _mlp_call(3)(
      h2, a3, p['w3a'], row(p['b3a']), p['w3b'], row(p['b3b']),
      row(p['bn3_g']), row(p['bn3_b']), p['pw3'],
      pw4[2 * D:], xw4b)
  y = _y_call(xw1, xw2, xw3, xw4c, dinv)
  nbr = agg(y, srcp, dstp)

  pb = jnp.concatenate([p['pb1'], p['pb2'], p['pb3'], p['pb4']])
  local, g4 = _pool_call(h1, h2, h3, y, nbr, dinv, row(ibatch),
                         row(pb))
  return (local, g4)


# speedup vs baseline: 7.2702x; 1.1656x over previous
import functools

import jax
import jax.numpy as jnp
from jax import lax
from jax.experimental import pallas as pl
from jax.experimental.pallas import tpu as pltpu
from jax.experimental.pallas import tpu_sc as plsc

N = 10000
D = 128
G = 64

NC = 2
NS = 16
LANES = 16
K = 128
NP = 10240
HALF = NP // 2
JROWS = 128
ACC_ROWS = HALF + JROWS
ZBLKS = ACC_ROWS // K
OUT_RPT = HALF // NS
PAD_ROW = N
NBUF = 2
MB = 1
MROWS = MB * K

_HIGH = jax.lax.Precision.HIGHEST
_f32 = jnp.float32


def _fill(ref, nrows, value):
  v = jnp.full((LANES,), value, _f32)

  @pl.loop(0, nrows)
  def _(r):
    for cg in range(D // LANES):
      ref[r, pl.ds(cg * LANES, LANES)] = v


def _zero_acc(acc, zsrc, s):
  for bi in range(-(-ZBLKS // NS)):
    b = bi * NS + s

    @pl.when(b < ZBLKS)
    def _():
      pltpu.sync_copy(zsrc, acc.at[pl.ds(b * K, K)])


def _make_edge_scatter(with_deg, nblk):
  mesh = plsc.VectorSubcoreMesh(core_axis_name="c", subcore_axis_name="s")
  out_type = jax.ShapeDtypeStruct((NP, D), _f32)
  if with_deg:
    out_type = [out_type, jax.ShapeDtypeStruct((NP, D), _f32)]
  CROWS = nblk + 8
  TRASH = CROWS - 1
  scratch = [
      pltpu.VMEM((CROWS, K), jnp.int32),
      pltpu.VMEM((CROWS, K), jnp.int32),
      pltpu.VMEM((NBUF, K, D), _f32),
      pltpu.VMEM_SHARED((ACC_ROWS, D), _f32),
      pltpu.SemaphoreType.DMA((NBUF,)),
      pltpu.SemaphoreType.DMA((NBUF,)),
  ]

  def body(feat, srcp, dstp, *rest):
    if with_deg:
      out, deg_out, srcc, dstc, rows, acc, gsem, ssem = rest
    else:
      out, srcc, dstc, rows, acc, gsem, ssem = rest
    c = lax.axis_index("c")
    s = lax.axis_index("s")

    pltpu.sync_copy(srcp.at[pl.ds(s * nblk, nblk)], srcc.at[pl.ds(0, nblk)])
    pltpu.sync_copy(dstp.at[pl.ds(s * nblk, nblk)], dstc.at[pl.ds(0, nblk)])

    base = c * HALF
    iota16 = lax.iota(jnp.int32, LANES)

    @pl.loop(0, nblk, init_carry=jnp.int32(0))
    def cnt(r, off):
      for cg in range(K // LANES):
        dv = dstc[r, pl.ds(cg * LANES, LANES)]
        sv = srcc[r, pl.ds(cg * LANES, LANES)]
        loc = dv - base
        m = (loc >= 0) & (loc < HALF)
        mi = m.astype(jnp.int32)
        pos = off + plsc.cumsum(mi) - mi
        rv = jnp.where(m, pos >> 7, TRASH)
        cv = jnp.where(m, pos & (K - 1), iota16)
        plsc.store_scatter(dstc, [rv, cv], loc)
        plsc.store_scatter(srcc, [rv, cv], sv)
        off = off + jnp.sum(mi)
      return off

    for g2 in range(2 * K // LANES):
      pos = cnt + g2 * LANES + iota16
      rv, cv = pos >> 7, pos & (K - 1)
      plsc.store_scatter(dstc, [rv, cv], HALF + iota16)
      plsc.store_scatter(srcc, [rv, cv], jnp.full((LANES,), PAD_ROW, jnp.int32))
    nbc2 = (cnt + jnp.int32(2 * K - 1)) // jnp.int32(2 * K)
    nbt = 2 * nbc2

    _fill(rows.at[0], K, 0.0)

    def drain(out_ref):
      plsc.subcore_barrier()
      pltpu.sync_copy(acc.at[pl.ds(s * OUT_RPT, OUT_RPT)],
                      out_ref.at[pl.ds(base + s * OUT_RPT, OUT_RPT)])
      plsc.subcore_barrier()

    def wait_scatter(src_b, sem_b, j):
      pltpu.make_async_copy(rows.at[src_b], acc.at[dstc.at[j]],
                            ssem.at[sem_b]).wait()

    if with_deg:
      _zero_acc(acc, rows.at[0], s)
      _fill(rows.at[1], K, 1.0)
      plsc.subcore_barrier()

      @pl.loop(0, nbc2)
      def _(g):
        for b in range(2):
          j = 2 * g + b

          @pl.when(j >= 2)
          def _():
            wait_scatter(1, b, j - 2)

          pltpu.async_copy(rows.at[1], acc.at[dstc.at[j]], ssem.at[b],
                           add=True)

      @pl.when(nbc2 > 0)
      def _():
        wait_scatter(1, 0, nbt - 2)
        wait_scatter(1, 1, nbt - 1)

      drain(deg_out)

    _zero_acc(acc, rows.at[0], s)
    plsc.subcore_barrier()

    @pl.when(nbc2 > 0)
    def _():
      pltpu.async_copy(feat.at[srcc.at[0]], rows.at[0], gsem.at[0])

    @pl.loop(0, nbc2)
    def _(g):
      for b in range(2):
        j = 2 * g + b
        bg = 1 - b

        @pl.when((j >= 1) & (j + 1 < nbt))
        def _():
          wait_scatter(bg, bg, j - 1)

        @pl.when(j + 1 < nbt)
        def _():
          pltpu.async_copy(feat.at[srcc.at[j + 1]], rows.at[bg], gsem.at[bg])

        pltpu.make_async_copy(feat.at[srcc.at[j]], rows.at[b],
                              gsem.at[b]).wait()
        pltpu.async_copy(rows.at[b], acc.at[dstc.at[j]], ssem.at[b], add=True)

    @pl.when(nbc2 > 0)
    def _():
      wait_scatter(0, 0, nbt - 2)
      wait_scatter(1, 1, nbt - 1)

    drain(out)

  return pl.kernel(
      body, out_type=out_type, mesh=mesh, scratch_types=scratch,
      compiler_params=pltpu.CompilerParams(needs_layout_passes=False))


def _bn(x, g, b):
  m = jnp.mean(x, axis=0, keepdims=True)
  v = jnp.mean((x - m) ** 2, axis=0, keepdims=True)
  return (x - m) / jnp.sqrt(v + 1e-5) * g + b


def _bn0_body(x_ref, g_ref, b_ref, o_ref):
  h = _bn(x_ref[...], g_ref[...], b_ref[...])
  o_ref[...] = jnp.concatenate([h, jnp.zeros((NP - N, D), _f32)], axis=0)


_bn0_call = pl.pallas_call(
    _bn0_body, out_shape=jax.ShapeDtypeStruct((NP, D), _f32))


def _mlp_body(layer, *refs):
  if layer == 1:
    (h_ref, a_ref, degp_ref, wa, ba, wb, bb, g_ref, b_ref, pw, pw4,
     h_out, xw_out, xw4_out, dinv_out) = refs
  else:
    (h_ref, a_ref, wa, ba, wb, bb, g_ref, b_ref, pw, pw4, xw4p_ref,
     h_out, xw_out, xw4_out) = refs

  t = h_ref[:N] + a_ref[:N]
  u = jnp.maximum(jnp.dot(t, wa[...]) + ba[...], 0.0)
  r = jnp.maximum(jnp.dot(u, wb[...]) + bb[...], 0.0)
  hn = _bn(r, g_ref[...], b_ref[...])
  h_out[...] = jnp.concatenate([hn, jnp.zeros((NP - N, D), _f32)], axis=0)

  zpad = jnp.zeros((NP - N,), _f32)
  xw = jnp.dot(hn, pw[...])[:, 0]
  xw_out[...] = jnp.concatenate([xw, zpad])[None, :]
  xw4 = jnp.dot(hn, pw4[...])[:, 0]
  if layer != 1:
    xw4 = xw4 + xw4p_ref[0, :N]
  xw4_out[...] = jnp.concatenate([xw4, zpad])[None, :]

  if layer == 1:
    deg = degp_ref[:N, 0] + 1.0
    dinv = deg ** -0.5
    dinv_out[...] = jnp.concatenate([dinv, zpad])[None, :]


def _mlp_call(layer):
  outs = [jax.ShapeDtypeStruct((NP, D), _f32),
          jax.ShapeDtypeStruct((1, NP), _f32),
          jax.ShapeDtypeStruct((1, NP), _f32)]
  if layer == 1:
    outs.append(jax.ShapeDtypeStruct((1, NP), _f32))
  return pl.pallas_call(functools.partial(_mlp_body, layer), out_shape=outs)


def _y_body(xw1_ref, xw2_ref, xw3_ref, xw4_ref, dinv_ref, y_out):
  dinv = dinv_ref[...]
  cols = [xw1_ref[...] * dinv, xw2_ref[...] * dinv,
          xw3_ref[...] * dinv, xw4_ref[...] * dinv]
  y = jnp.concatenate(cols, axis=0).T
  y_out[...] = jnp.concatenate([y, jnp.zeros((NP, D - 4), _f32)], axis=1)


_y_call = pl.pallas_call(
    _y_body, out_shape=jax.ShapeDtypeStruct((NP, D), _f32))


def _pool_body(h1_ref, h2_ref, h3_ref, y_ref, nbr_ref, dinv_ref, batch_ref,
               pb_ref, local_out, g4_out):
  dinv = dinv_ref[0, :N]
  gi = lax.broadcasted_iota(jnp.int32, (G, N), 0)
  ind = (batch_ref[...] == gi).astype(_f32)
  hs = (h1_ref[:N], h2_ref[:N], h3_ref[:N])

  def attn(k):
    sc = dinv * nbr_ref[:N, k] + y_ref[:N, k] * dinv + pb_ref[0, k]
    s = jnp.tanh(sc)
    smax = jnp.max(jnp.where(ind > 0, s[None, :], -1e30), axis=1)
    smb = jnp.dot(smax[None, :], ind, precision=_HIGH)[0]
    e = jnp.exp(s - smb)
    z = jnp.dot(ind, e[:, None], precision=_HIGH)[:, 0]
    zb = jnp.dot(z[None, :], ind, precision=_HIGH)[0]
    return e / (zb + 1e-16)

  gs = []
  for k in range(3):
    a = attn(k)
    gs.append(jnp.dot(ind, hs[k] * a[:, None], precision=_HIGH))
  local_out[...] = jnp.concatenate(gs, axis=1)

  a4 = attn(3)
  g4 = [jnp.dot(ind, h * a4[:, None], precision=_HIGH) for h in hs]
  g4_out[...] = jnp.concatenate(g4, axis=1)


_pool_call = pl.pallas_call(
    _pool_body,
    out_shape=[jax.ShapeDtypeStruct((G, 3 * D), _f32),
               jax.ShapeDtypeStruct((G, 3 * D), _f32)])


def kernel(drug_feature, drug_adj, ibatch, params):
  p = params
  src = drug_adj[0].astype(jnp.int32)
  dst = drug_adj[1].astype(jnp.int32)
  e = src.shape[0]
  nblk = -(-e // (NS * K))
  nblk = -(-nblk // 8) * 8
  ep = NS * nblk * K
  pad = jnp.full((ep - e,), PAD_ROW, jnp.int32)
  srcp = jnp.concatenate([src, pad]).reshape(NS * nblk // MB, MROWS)
  dstp = jnp.concatenate([dst, pad]).reshape(NS * nblk // MB, MROWS)

  agg_deg = _make_edge_scatter(True, nblk)
  agg = _make_edge_scatter(False, nblk)

  row = lambda a: a.reshape(1, -1)
  pw4 = p['pw4']

  h0 = _bn0_call(drug_feature, row(p['bn0_g']), row(p['bn0_b']))
  a1, degp = agg_deg(h0, srcp, dstp)
  h1, xw1, xw4a, dinv = _mlp_call(1)(
      h0, a1, degp, p['w1a'], row(p['b1a']), p['w1b'], row(p['b1b']),
      row(p['bn1_g']), row(p['bn1_b']), p['pw1'],
      pw4[:D])
  a2 = agg(h1, srcp, dstp)
  h2, xw2, xw4b = _mlp_call(2)(
      h1, a2, p['w2a'], row(p['b2a']), p['w2b'], row(p['b2b']),
      row(p['bn2_g']), row(p['bn2_b']), p['pw2'],
      pw4[D:2 * D], xw4a)
  a3 = agg(h2, srcp, dstp)
  h3, xw3, xw4c = _mlp_call(3)(
      h2, a3, p['w3a'], row(p['b3a']), p['w3b'], row(p['b3b']),
      row(p['bn3_g']), row(p['bn3_b']), p['pw3'],
      pw4[2 * D:], xw4b)
  y = _y_call(xw1, xw2, xw3, xw4c, dinv)
  nbr = agg(y, srcp, dstp)

  pb = jnp.concatenate([p['pb1'], p['pb2'], p['pb3'], p['pb4']])
  local, g4 = _pool_call(h1, h2, h3, y, nbr, dinv, row(ibatch),
                         row(pb))
  return (local, g4)
